# baseline (speedup 1.0000x reference)
import functools

import jax
import jax.numpy as jnp
from jax import lax
from jax.experimental import pallas as pl
from jax.experimental.pallas import tpu as pltpu
from jax.experimental.pallas import tpu_sc as plsc

BATCH = 16384
HIST = 20
SKU_DIM = 64
CAT_DIM = 16
PRICE_DIM = 16
WORD_DIM = 64
ITEM_DIM = 128
CONCAT_DIM = SKU_DIM + CAT_DIM + PRICE_DIM + WORD_DIM

NUM_CORES = 2
NUM_SUBCORES = 16
NW = NUM_CORES * NUM_SUBCORES
BPW = BATCH // NW
IDX_CHUNK = 128
WCHUNK = 16
WROWS = WCHUNK * HIST
NWCHUNK = BPW // WCHUNK
WSPLIT = ((0, 128), (128, 128), (256, 64))


def _sc_scp(sku_id, cat_id, price_id, sku_table, cat_table, price_table):
    mesh = plsc.VectorSubcoreMesh(core_axis_name="c", subcore_axis_name="s")
    f32 = jnp.float32

    @functools.partial(
        pl.kernel,
        out_type=[
            jax.ShapeDtypeStruct((BATCH, SKU_DIM), f32),
            jax.ShapeDtypeStruct((BATCH, CAT_DIM), f32),
            jax.ShapeDtypeStruct((BATCH, PRICE_DIM), f32),
        ],
        mesh=mesh,
        compiler_params=pltpu.CompilerParams(use_tc_tiling_on_sc=False),
        scratch_types=[
            pltpu.VMEM((BPW,), jnp.int32),
            pltpu.VMEM((BPW,), jnp.int32),
            pltpu.VMEM((BPW,), jnp.int32),
            pltpu.VMEM((BPW, SKU_DIM), f32),
            pltpu.VMEM((BPW, CAT_DIM), f32),
            pltpu.VMEM((BPW, PRICE_DIM), f32),
            pltpu.SemaphoreType.DMA,
            pltpu.SemaphoreType.DMA,
            pltpu.SemaphoreType.DMA,
        ],
    )
    def k(sku_id_h, cat_id_h, price_id_h,
          sku_t_h, cat_t_h, price_t_h,
          sku_o, cat_o, price_o,
          idx_s, idx_c, idx_p, sku_rows, cat_rows, price_rows,
          sem_i, sem_g, sem_o):
        wid = lax.axis_index("s") * NUM_CORES + lax.axis_index("c")
        base = wid * BPW

        ic = [
            pltpu.async_copy(sku_id_h.at[pl.ds(base, BPW)], idx_s, sem_i),
            pltpu.async_copy(cat_id_h.at[pl.ds(base, BPW)], idx_c, sem_i),
            pltpu.async_copy(price_id_h.at[pl.ds(base, BPW)], idx_p, sem_i),
        ]
        for c in ic:
            c.wait()

        for j in range(BPW // IDX_CHUNK):
            sl = pl.ds(j * IDX_CHUNK, IDX_CHUNK)
            pltpu.async_copy(sku_t_h.at[idx_s.at[sl]], sku_rows.at[sl], sem_g)
            pltpu.async_copy(cat_t_h.at[idx_c.at[sl]], cat_rows.at[sl], sem_g)
            pltpu.async_copy(price_t_h.at[idx_p.at[sl]], price_rows.at[sl],
                             sem_g)
        for j in range(BPW // IDX_CHUNK):
            sl = pl.ds(j * IDX_CHUNK, IDX_CHUNK)
            pltpu.make_async_copy(sku_t_h.at[pl.ds(0, IDX_CHUNK)],
                                  sku_rows.at[sl], sem_g).wait()
            pltpu.make_async_copy(cat_t_h.at[pl.ds(0, IDX_CHUNK)],
                                  cat_rows.at[sl], sem_g).wait()
            pltpu.make_async_copy(price_t_h.at[pl.ds(0, IDX_CHUNK)],
                                  price_rows.at[sl], sem_g).wait()
        out_sl = pl.ds(base, BPW)
        oc = [
            pltpu.async_copy(sku_rows, sku_o.at[out_sl], sem_o),
            pltpu.async_copy(cat_rows, cat_o.at[out_sl], sem_o),
            pltpu.async_copy(price_rows, price_o.at[out_sl], sem_o),
        ]

        for c in oc:
            c.wait()

    return k(sku_id, cat_id, price_id, sku_table, cat_table, price_table)


def _sc_word(wids, word_table):
    mesh = plsc.VectorSubcoreMesh(core_axis_name="c", subcore_axis_name="s")
    f32 = jnp.float32

    @functools.partial(
        pl.kernel,
        out_type=[jax.ShapeDtypeStruct((BATCH, WORD_DIM), f32)],
        mesh=mesh,
        compiler_params=pltpu.CompilerParams(use_tc_tiling_on_sc=False),
        scratch_types=[
            pltpu.VMEM((BPW * HIST,), jnp.int32),
            pltpu.VMEM((2, WROWS, WORD_DIM), f32),
            pltpu.VMEM((2, WCHUNK, WORD_DIM), f32),
            pltpu.SemaphoreType.DMA,
            pltpu.SemaphoreType.DMA,
            pltpu.SemaphoreType.DMA,
            pltpu.SemaphoreType.DMA,
            pltpu.SemaphoreType.DMA,
        ],
    )
    def k(wids_h, word_t_h, word_o,
          idx_w, wbuf, mean_buf, sem_i, sem_w0, sem_w1, sem_m0, sem_m1):
        wid = lax.axis_index("s") * NUM_CORES + lax.axis_index("c")
        base = wid * BPW
        wbase = base * HIST
        sem_w = (sem_w0, sem_w1)
        sem_m = (sem_m0, sem_m1)

        pltpu.async_copy(wids_h.at[pl.ds(wbase, BPW * HIST)], idx_w,
                         sem_i).wait()

        def fire_wchunk(chunk, par):
            for off, n in WSPLIT:
                pltpu.async_copy(
                    word_t_h.at[idx_w.at[pl.ds(chunk * WROWS + off, n)]],
                    wbuf.at[par, pl.ds(off, n)], sem_w[par])

        def drain_wchunk(par):
            for off, n in WSPLIT:
                pltpu.make_async_copy(
                    word_t_h.at[pl.ds(0, n)],
                    wbuf.at[par, pl.ds(off, n)], sem_w[par]).wait()

        fire_wchunk(0, 0)
        fire_wchunk(1, 1)

        @pl.loop(0, NWCHUNK, step=2)
        def _c(c):
            for par in range(2):
                chunk = c + par
                drain_wchunk(par)

                @pl.when(chunk >= 2)
                def _():
                    pltpu.make_async_copy(
                        word_t_h.at[pl.ds(0, WCHUNK)],
                        mean_buf.at[par], sem_m[par]).wait()

                @pl.loop(0, WCHUNK)
                def _sample(s):
                    r0 = s * HIST
                    for d in range(WORD_DIM // 16):
                        sl = pl.ds(d * 16, 16)
                        acc = wbuf[par, r0, sl]
                        for h in range(1, HIST):
                            acc = acc + wbuf[par, r0 + h, sl]
                        mean_buf[par, s, sl] = acc * (1.0 / HIST)

                pltpu.async_copy(
                    mean_buf.at[par],
                    word_o.at[pl.ds(base + chunk * WCHUNK, WCHUNK)],
                    sem_m[par])

                @pl.when(chunk + 2 < NWCHUNK)
                def _():
                    fire_wchunk(chunk + 2, par)

        for par in range(2):
            pltpu.make_async_copy(word_t_h.at[pl.ds(0, WCHUNK)],
                                  mean_buf.at[par], sem_m[par]).wait()

    return k(wids, word_table)[0]


def _tc_dense(sku_emb, cat_emb, price_emb, word_emb, W, b):
    BM = 1024

    def body(s_ref, c_ref, p_ref, w_ref, W_ref, b_ref, o_ref):
        x = jnp.concatenate(
            [s_ref[...], c_ref[...], p_ref[...], w_ref[...]], axis=-1)
        acc = jnp.dot(x, W_ref[...], preferred_element_type=jnp.float32)
        o_ref[...] = jnp.maximum(acc + b_ref[...], 0.0)

    return pl.pallas_call(
        body,
        grid=(BATCH // BM,),
        in_specs=[
            pl.BlockSpec((BM, SKU_DIM), lambda i: (i, 0)),
            pl.BlockSpec((BM, CAT_DIM), lambda i: (i, 0)),
            pl.BlockSpec((BM, PRICE_DIM), lambda i: (i, 0)),
            pl.BlockSpec((BM, WORD_DIM), lambda i: (i, 0)),
            pl.BlockSpec((CONCAT_DIM, ITEM_DIM), lambda i: (0, 0)),
            pl.BlockSpec((1, ITEM_DIM), lambda i: (0, 0)),
        ],
        out_specs=pl.BlockSpec((BM, ITEM_DIM), lambda i: (i, 0)),
        out_shape=jax.ShapeDtypeStruct((BATCH, ITEM_DIM), jnp.float32),
    )(sku_emb, cat_emb, price_emb, word_emb, W, b)


def kernel(sku_id, cat_id, price_id, word_ids, sku_table, cat_table,
           price_table, word_table, W, b):
    sku_id = sku_id.astype(jnp.int32)
    cat_id = cat_id.astype(jnp.int32)
    price_id = price_id.astype(jnp.int32)
    wids = word_ids.astype(jnp.int32).reshape(-1)
    sku_emb, cat_emb, price_emb = _sc_scp(
        sku_id, cat_id, price_id, sku_table, cat_table, price_table)
    word_emb = _sc_word(wids, word_table)
    return _tc_dense(sku_emb, cat_emb, price_emb, word_emb,
                     W, b.reshape(1, ITEM_DIM))
